# baseline (speedup 1.0000x reference)
import functools

import jax
import jax.numpy as jnp
from jax import lax
from jax.experimental import pallas as pl
from jax.experimental.pallas import tpu as pltpu
from jax.experimental.pallas import tpu_sc as plsc

N = 10000
E = 320000
IN_CH = 128
HID_CH = 256
OUT_CH = 64

NC = 2
NS = 16
NW = NC * NS

K1 = 125
C1 = E // NW // K1
K2 = 125
C2 = E // NS // K2
NP = 10240
ROWS_T = NP // NS

def _edge_pipeline(C, ld4, ig, wg, isc, wsc):
    M = C // 4
    assert C % 8 == 0 and M >= 4
    ld4(0, 0)
    ig(0, 0, 0)
    ig(1, 0, 1)
    wg(0, 0, 0)
    isc(0, 0, 0)
    wsc(0, 0, 0)
    ld4(1, 1)
    ig(0, 0, 2)
    wg(1, 0, 1)
    isc(1, 0, 1)
    wsc(1, 0, 1)
    ig(1, 0, 3)
    wg(0, 0, 2)
    isc(0, 0, 2)

    def superstep(m, p, prefetch):
        o = 1 - p
        wsc(0, o, 2)
        ig(0, p, 0)
        wg(1, o, 3)
        isc(1, o, 3)
        wsc(1, o, 3)
        ig(1, p, 1)
        wg(0, p, 0)
        isc(0, p, 0)
        wsc(0, p, 0)
        if prefetch:
            ld4(m + 1, o)
        ig(0, p, 2)
        wg(1, p, 1)
        isc(1, p, 1)
        wsc(1, p, 1)
        ig(1, p, 3)
        wg(0, p, 2)
        isc(0, p, 2)

    def body(t, carry):
        superstep(2 * t + 1, 1, True)
        superstep(2 * t + 2, 0, True)
        return carry

    lax.fori_loop(0, (M - 2) // 2, body, 0)
    superstep(M - 1, 1, False)
    wg(1, 1, 3)
    isc(1, 1, 3)
    wsc(0, 1, 2)
    wsc(1, 1, 3)


@functools.lru_cache(maxsize=None)
def _sc_kernels():
    mesh = plsc.VectorSubcoreMesh(core_axis_name="c", subcore_axis_name="s",
                                  num_cores=NC, num_subcores=NS)

    @functools.partial(
        pl.kernel,
        out_type=(
            jax.ShapeDtypeStruct((NC, NP, IN_CH), jnp.float32),
            jax.ShapeDtypeStruct((NC, NP), jnp.float32),
        ),
        mesh=mesh,
        scratch_types=[
            pltpu.VMEM((4, 2, K1), jnp.int32),
            pltpu.VMEM((4, 2, K1), jnp.int32),
            pltpu.VMEM((K1, IN_CH), jnp.float32),
            pltpu.VMEM((K1, IN_CH), jnp.float32),
            pltpu.VMEM((128,), jnp.float32),
            pltpu.VMEM_SHARED((NP, IN_CH), jnp.float32),
            pltpu.VMEM_SHARED((NP,), jnp.float32),
            pltpu.SemaphoreType.DMA,
            pltpu.SemaphoreType.DMA,
            pltpu.SemaphoreType.DMA,
            pltpu.SemaphoreType.DMA,
        ],
    )
    def sc_layer1(x_hbm, sd4, zf, z1, aggp, cntp,
                  ib0, ib1, rv0, rv1, onesv, acc, cacc,
                  gs0, gs1, ss0, ss1):
        c = lax.axis_index("c")
        s = lax.axis_index("s")
        w = c * NS + s
        r0 = s * ROWS_T
        ib = (ib0, ib1)
        rv = (rv0, rv1)
        gs = (gs0, gs1)
        ss = (ss0, ss1)
        pltpu.sync_copy(zf, acc.at[pl.ds(r0, ROWS_T)])
        pltpu.sync_copy(z1, cacc.at[pl.ds(r0, ROWS_T)])
        ones16 = jnp.ones((16,), jnp.float32)
        for i in range(8):
            onesv[pl.ds(i * 16, 16)] = ones16
        plsc.subcore_barrier()

        def ld4(m, p):
            pltpu.sync_copy(sd4.at[w].at[m], ib[p])

        def ig(b, p, r):
            pltpu.async_copy(x_hbm.at[ib[p].at[r].at[0]], rv[b], gs[b])

        def wg(b, p, r):
            pltpu.make_async_copy(x_hbm.at[ib[p].at[r].at[0]], rv[b],
                                  gs[b]).wait()

        def isc(b, p, r):
            pltpu.async_copy(rv[b], acc.at[ib[p].at[r].at[1]], ss[b],
                             add=True)
            pltpu.async_copy(onesv.at[pl.ds(0, K1)],
                             cacc.at[ib[p].at[r].at[1]], ss[b], add=True)

        def wsc(b, p, r):
            pltpu.make_async_copy(rv[b], acc.at[ib[p].at[r].at[1]],
                                  ss[b]).wait()
            pltpu.make_async_copy(onesv.at[pl.ds(0, K1)],
                                  cacc.at[ib[p].at[r].at[1]], ss[b]).wait()

        _edge_pipeline(C1, ld4, ig, wg, isc, wsc)
        plsc.subcore_barrier()
        pltpu.sync_copy(acc.at[pl.ds(r0, ROWS_T)],
                        aggp.at[c].at[pl.ds(r0, ROWS_T)])
        pltpu.sync_copy(cacc.at[pl.ds(r0, ROWS_T)],
                        cntp.at[c].at[pl.ds(r0, ROWS_T)])

    @functools.partial(
        pl.kernel,
        out_type=jax.ShapeDtypeStruct((NC, NP, IN_CH), jnp.float32),
        mesh=mesh,
        scratch_types=[
            pltpu.VMEM((4, 2, K2), jnp.int32),
            pltpu.VMEM((4, 2, K2), jnp.int32),
            pltpu.VMEM((K2, IN_CH), jnp.float32),
            pltpu.VMEM((K2, IN_CH), jnp.float32),
            pltpu.VMEM_SHARED((NP, IN_CH), jnp.float32),
            pltpu.SemaphoreType.DMA,
            pltpu.SemaphoreType.DMA,
            pltpu.SemaphoreType.DMA,
            pltpu.SemaphoreType.DMA,
        ],
    )
    def sc_layer2(h1s_hbm, sdB, zf, agg2,
                  ib0, ib1, rv0, rv1, acc, gs0, gs1, ss0, ss1):
        c = lax.axis_index("c")
        s = lax.axis_index("s")
        r0 = s * ROWS_T
        ib = (ib0, ib1)
        rv = (rv0, rv1)
        gs = (gs0, gs1)
        ss = (ss0, ss1)
        pltpu.sync_copy(zf, acc.at[pl.ds(r0, ROWS_T)])
        plsc.subcore_barrier()

        def ld4(m, p):
            pltpu.sync_copy(sdB.at[s].at[m], ib[p])

        def ig(b, p, r):
            pltpu.async_copy(h1s_hbm.at[c].at[ib[p].at[r].at[0]], rv[b],
                             gs[b])

        def wg(b, p, r):
            pltpu.make_async_copy(h1s_hbm.at[c].at[ib[p].at[r].at[0]],
                                  rv[b], gs[b]).wait()

        def isc(b, p, r):
            pltpu.async_copy(rv[b], acc.at[ib[p].at[r].at[1]], ss[b],
                             add=True)

        def wsc(b, p, r):
            pltpu.make_async_copy(rv[b], acc.at[ib[p].at[r].at[1]],
                                  ss[b]).wait()

        _edge_pipeline(C2, ld4, ig, wg, isc, wsc)
        plsc.subcore_barrier()
        pltpu.sync_copy(acc.at[pl.ds(r0, ROWS_T)],
                        agg2.at[c].at[pl.ds(r0, ROWS_T)])

    return sc_layer1, sc_layer2


BM = 2000
_GM = N // BM
_PREC = lax.Precision.HIGHEST


def _root_body(x, wr, bl, xr):
    xr[...] = lax.dot(x[...], wr[...], precision=_PREC) + bl[...]


def _root2_body(h1s, wr, bl, hr):
    h1 = jnp.concatenate([h1s[0], h1s[1]], axis=1)
    hr[...] = lax.dot(h1, wr[...], precision=_PREC) + bl[...]


def _d1_body(aggp, rcnt, xr, wl, h1s):
    agg = aggp[0] + aggp[1]
    mw = lax.dot(agg, wl[...], precision=_PREC) * rcnt[...]
    h = jnp.maximum(mw + xr[...], 0.0)
    h1s[0] = h[:, :IN_CH]
    h1s[1] = h[:, IN_CH:]


def _d2_body(agg2, rcnt, hr, wl, wfc, bfc, out):
    agg = jnp.concatenate([agg2[0], agg2[1]], axis=1)
    mw = lax.dot(agg, wl[...], precision=_PREC) * rcnt[...]
    h2 = jnp.maximum(mw + hr[...], 0.0)
    z = lax.dot(h2, wfc[...], precision=_PREC) + bfc[...]
    m = jnp.max(z, axis=1, keepdims=True)
    e = z - m
    out[...] = e - jnp.log(jnp.sum(jnp.exp(e), axis=1, keepdims=True))


_root_call = pl.pallas_call(
    _root_body,
    grid=(_GM,),
    in_specs=[
        pl.BlockSpec((BM, IN_CH), lambda i: (i, 0)),
        pl.BlockSpec((IN_CH, HID_CH), lambda i: (0, 0)),
        pl.BlockSpec((1, HID_CH), lambda i: (0, 0)),
    ],
    out_specs=pl.BlockSpec((BM, HID_CH), lambda i: (i, 0)),
    out_shape=jax.ShapeDtypeStruct((N, HID_CH), jnp.float32),
)

_root2_call = pl.pallas_call(
    _root2_body,
    grid=(_GM,),
    in_specs=[
        pl.BlockSpec((2, BM, IN_CH), lambda i: (0, i, 0)),
        pl.BlockSpec((HID_CH, HID_CH), lambda i: (0, 0)),
        pl.BlockSpec((1, HID_CH), lambda i: (0, 0)),
    ],
    out_specs=pl.BlockSpec((BM, HID_CH), lambda i: (i, 0)),
    out_shape=jax.ShapeDtypeStruct((N, HID_CH), jnp.float32),
)

_d1_call = pl.pallas_call(
    _d1_body,
    grid=(_GM,),
    in_specs=[
        pl.BlockSpec((2, BM, IN_CH), lambda i: (0, i, 0)),
        pl.BlockSpec((BM, 1), lambda i: (i, 0)),
        pl.BlockSpec((BM, HID_CH), lambda i: (i, 0)),
        pl.BlockSpec((IN_CH, HID_CH), lambda i: (0, 0)),
    ],
    out_specs=pl.BlockSpec((2, BM, IN_CH), lambda i: (0, i, 0)),
    out_shape=jax.ShapeDtypeStruct((2, N, IN_CH), jnp.float32),
)

_d2_call = pl.pallas_call(
    _d2_body,
    grid=(_GM,),
    in_specs=[
        pl.BlockSpec((2, BM, IN_CH), lambda i: (0, i, 0)),
        pl.BlockSpec((BM, 1), lambda i: (i, 0)),
        pl.BlockSpec((BM, HID_CH), lambda i: (i, 0)),
        pl.BlockSpec((HID_CH, HID_CH), lambda i: (0, 0)),
        pl.BlockSpec((HID_CH, OUT_CH), lambda i: (0, 0)),
        pl.BlockSpec((1, OUT_CH), lambda i: (0, 0)),
    ],
    out_specs=pl.BlockSpec((BM, OUT_CH), lambda i: (i, 0)),
    out_shape=jax.ShapeDtypeStruct((N, OUT_CH), jnp.float32),
)


def kernel(x, edge_index, Wl1, bl1, Wr1, Wl2, bl2, Wr2, Wfc, bfc):
    src = edge_index[0].astype(jnp.int32)
    dst = edge_index[1].astype(jnp.int32)
    sd = jnp.stack([src.reshape(-1, K1), dst.reshape(-1, K1)], axis=1)
    sd3 = sd.reshape(NW, C1 // 4, 4, 2, K1)
    sdB = sd.reshape(NS, C2 // 4, 4, 2, K2)
    zf = jnp.zeros((ROWS_T, IN_CH), jnp.float32)
    z1 = jnp.zeros((ROWS_T,), jnp.float32)

    sc_layer1, sc_layer2 = _sc_kernels()
    xr = _root_call(x, Wr1, bl1.reshape(1, -1))
    aggp, cntp = sc_layer1(x, sd3, zf, z1)
    rcnt = (1.0 / jnp.clip(cntp[0] + cntp[1], 1.0, None))[:, None]
    h1s = _d1_call(aggp, rcnt, xr, Wl1)
    hr = _root2_call(h1s, Wr2, bl2.reshape(1, -1))
    agg2 = sc_layer2(h1s, sdB, zf)
    out = _d2_call(agg2, rcnt, hr, Wl2, Wfc, bfc.reshape(1, -1))
    return out


# speedup vs baseline: 1.0716x; 1.0716x over previous
import functools

import jax
import jax.numpy as jnp
from jax import lax
from jax.experimental import pallas as pl
from jax.experimental.pallas import tpu as pltpu
from jax.experimental.pallas import tpu_sc as plsc

N = 10000
E = 320000
IN_CH = 128
HID_CH = 256
OUT_CH = 64

NC = 2
NS = 16
NW = NC * NS

K1 = 125
C1 = E // NW // K1
K2 = 125
C2 = E // NS // K2
NP = 10240
ROWS_T = NP // NS

def _edge_pipeline(C, ld4, wl4, ig, wg, isc, wsc):
    M = C // 4
    assert C % 8 == 0 and M >= 4
    ld4(0, 0)
    wl4(0)
    ig(0, 0, 0)
    ig(1, 0, 1)
    wg(0, 0, 0)
    isc(0, 0, 0)
    wsc(0, 0, 0)
    ld4(1, 1)
    ig(0, 0, 2)
    wg(1, 0, 1)
    isc(1, 0, 1)
    wsc(1, 0, 1)
    ig(1, 0, 3)
    wg(0, 0, 2)
    isc(0, 0, 2)

    def superstep(m, p, prefetch):
        o = 1 - p
        wl4(p)
        wsc(0, o, 2)
        ig(0, p, 0)
        wg(1, o, 3)
        isc(1, o, 3)
        wsc(1, o, 3)
        ig(1, p, 1)
        wg(0, p, 0)
        isc(0, p, 0)
        wsc(0, p, 0)
        if prefetch:
            ld4(m + 1, o)
        ig(0, p, 2)
        wg(1, p, 1)
        isc(1, p, 1)
        wsc(1, p, 1)
        ig(1, p, 3)
        wg(0, p, 2)
        isc(0, p, 2)

    def body(t, carry):
        superstep(2 * t + 1, 1, True)
        superstep(2 * t + 2, 0, True)
        return carry

    lax.fori_loop(0, (M - 2) // 2, body, 0)
    superstep(M - 1, 1, False)
    wg(1, 1, 3)
    isc(1, 1, 3)
    wsc(0, 1, 2)
    wsc(1, 1, 3)


@functools.lru_cache(maxsize=None)
def _sc_kernels():
    mesh = plsc.VectorSubcoreMesh(core_axis_name="c", subcore_axis_name="s",
                                  num_cores=NC, num_subcores=NS)

    @functools.partial(
        pl.kernel,
        out_type=(
            jax.ShapeDtypeStruct((NC, NP, IN_CH), jnp.float32),
            jax.ShapeDtypeStruct((NC, NP), jnp.float32),
        ),
        mesh=mesh,
        scratch_types=[
            pltpu.VMEM((4, 2, K1), jnp.int32),
            pltpu.VMEM((4, 2, K1), jnp.int32),
            pltpu.VMEM((K1, IN_CH), jnp.float32),
            pltpu.VMEM((K1, IN_CH), jnp.float32),
            pltpu.VMEM((128,), jnp.float32),
            pltpu.VMEM_SHARED((NP, IN_CH), jnp.float32),
            pltpu.VMEM_SHARED((NP,), jnp.float32),
            pltpu.SemaphoreType.DMA,
            pltpu.SemaphoreType.DMA,
            pltpu.SemaphoreType.DMA,
            pltpu.SemaphoreType.DMA,
            pltpu.SemaphoreType.DMA,
            pltpu.SemaphoreType.DMA,
        ],
    )
    def sc_layer1(x_hbm, sd4, zf, z1, aggp, cntp,
                  ib0, ib1, rv0, rv1, onesv, acc, cacc,
                  gs0, gs1, ss0, ss1, ip0, ip1):
        c = lax.axis_index("c")
        s = lax.axis_index("s")
        w = c * NS + s
        r0 = s * ROWS_T
        ib = (ib0, ib1)
        rv = (rv0, rv1)
        gs = (gs0, gs1)
        ss = (ss0, ss1)
        pltpu.sync_copy(zf, acc.at[pl.ds(r0, ROWS_T)])
        pltpu.sync_copy(z1, cacc.at[pl.ds(r0, ROWS_T)])
        ones16 = jnp.ones((16,), jnp.float32)
        for i in range(8):
            onesv[pl.ds(i * 16, 16)] = ones16
        plsc.subcore_barrier()

        ip = (ip0, ip1)

        def ld4(m, p):
            pltpu.async_copy(sd4.at[w].at[m], ib[p], ip[p])

        def wl4(p):
            pltpu.make_async_copy(sd4.at[w].at[0], ib[p], ip[p]).wait()

        def ig(b, p, r):
            pltpu.async_copy(x_hbm.at[ib[p].at[r].at[0]], rv[b], gs[b])

        def wg(b, p, r):
            pltpu.make_async_copy(x_hbm.at[ib[p].at[r].at[0]], rv[b],
                                  gs[b]).wait()

        def isc(b, p, r):
            pltpu.async_copy(rv[b], acc.at[ib[p].at[r].at[1]], ss[b],
                             add=True)
            pltpu.async_copy(onesv.at[pl.ds(0, K1)],
                             cacc.at[ib[p].at[r].at[1]], ss[b], add=True)

        def wsc(b, p, r):
            pltpu.make_async_copy(rv[b], acc.at[ib[p].at[r].at[1]],
                                  ss[b]).wait()
            pltpu.make_async_copy(onesv.at[pl.ds(0, K1)],
                                  cacc.at[ib[p].at[r].at[1]], ss[b]).wait()

        _edge_pipeline(C1, ld4, wl4, ig, wg, isc, wsc)
        plsc.subcore_barrier()
        pltpu.sync_copy(acc.at[pl.ds(r0, ROWS_T)],
                        aggp.at[c].at[pl.ds(r0, ROWS_T)])
        pltpu.sync_copy(cacc.at[pl.ds(r0, ROWS_T)],
                        cntp.at[c].at[pl.ds(r0, ROWS_T)])

    @functools.partial(
        pl.kernel,
        out_type=jax.ShapeDtypeStruct((NC, NP, IN_CH), jnp.float32),
        mesh=mesh,
        scratch_types=[
            pltpu.VMEM((4, 2, K2), jnp.int32),
            pltpu.VMEM((4, 2, K2), jnp.int32),
            pltpu.VMEM((K2, IN_CH), jnp.float32),
            pltpu.VMEM((K2, IN_CH), jnp.float32),
            pltpu.VMEM_SHARED((NP, IN_CH), jnp.float32),
            pltpu.SemaphoreType.DMA,
            pltpu.SemaphoreType.DMA,
            pltpu.SemaphoreType.DMA,
            pltpu.SemaphoreType.DMA,
            pltpu.SemaphoreType.DMA,
            pltpu.SemaphoreType.DMA,
        ],
    )
    def sc_layer2(h1s_hbm, sdB, zf, agg2,
                  ib0, ib1, rv0, rv1, acc, gs0, gs1, ss0, ss1, ip0, ip1):
        c = lax.axis_index("c")
        s = lax.axis_index("s")
        r0 = s * ROWS_T
        ib = (ib0, ib1)
        rv = (rv0, rv1)
        gs = (gs0, gs1)
        ss = (ss0, ss1)
        pltpu.sync_copy(zf, acc.at[pl.ds(r0, ROWS_T)])
        plsc.subcore_barrier()

        ip = (ip0, ip1)

        def ld4(m, p):
            pltpu.async_copy(sdB.at[s].at[m], ib[p], ip[p])

        def wl4(p):
            pltpu.make_async_copy(sdB.at[s].at[0], ib[p], ip[p]).wait()

        def ig(b, p, r):
            pltpu.async_copy(h1s_hbm.at[c].at[ib[p].at[r].at[0]], rv[b],
                             gs[b])

        def wg(b, p, r):
            pltpu.make_async_copy(h1s_hbm.at[c].at[ib[p].at[r].at[0]],
                                  rv[b], gs[b]).wait()

        def isc(b, p, r):
            pltpu.async_copy(rv[b], acc.at[ib[p].at[r].at[1]], ss[b],
                             add=True)

        def wsc(b, p, r):
            pltpu.make_async_copy(rv[b], acc.at[ib[p].at[r].at[1]],
                                  ss[b]).wait()

        _edge_pipeline(C2, ld4, wl4, ig, wg, isc, wsc)
        plsc.subcore_barrier()
        pltpu.sync_copy(acc.at[pl.ds(r0, ROWS_T)],
                        agg2.at[c].at[pl.ds(r0, ROWS_T)])

    return sc_layer1, sc_layer2


BM = 2000
_GM = N // BM
_PREC = lax.Precision.HIGHEST


def _root_body(x, wr, bl, xr):
    xr[...] = lax.dot(x[...], wr[...], precision=_PREC) + bl[...]


def _root2_body(h1s, wr, bl, hr):
    h1 = jnp.concatenate([h1s[0], h1s[1]], axis=1)
    hr[...] = lax.dot(h1, wr[...], precision=_PREC) + bl[...]


def _d1_body(aggp, rcnt, xr, wl, h1s):
    agg = aggp[0] + aggp[1]
    mw = lax.dot(agg, wl[...], precision=_PREC) * rcnt[...]
    h = jnp.maximum(mw + xr[...], 0.0)
    h1s[0] = h[:, :IN_CH]
    h1s[1] = h[:, IN_CH:]


def _d2_body(agg2, rcnt, hr, wl, wfc, bfc, out):
    agg = jnp.concatenate([agg2[0], agg2[1]], axis=1)
    mw = lax.dot(agg, wl[...], precision=_PREC) * rcnt[...]
    h2 = jnp.maximum(mw + hr[...], 0.0)
    z = lax.dot(h2, wfc[...], precision=_PREC) + bfc[...]
    m = jnp.max(z, axis=1, keepdims=True)
    e = z - m
    out[...] = e - jnp.log(jnp.sum(jnp.exp(e), axis=1, keepdims=True))


_root_call = pl.pallas_call(
    _root_body,
    grid=(_GM,),
    in_specs=[
        pl.BlockSpec((BM, IN_CH), lambda i: (i, 0)),
        pl.BlockSpec((IN_CH, HID_CH), lambda i: (0, 0)),
        pl.BlockSpec((1, HID_CH), lambda i: (0, 0)),
    ],
    out_specs=pl.BlockSpec((BM, HID_CH), lambda i: (i, 0)),
    out_shape=jax.ShapeDtypeStruct((N, HID_CH), jnp.float32),
)

_root2_call = pl.pallas_call(
    _root2_body,
    grid=(_GM,),
    in_specs=[
        pl.BlockSpec((2, BM, IN_CH), lambda i: (0, i, 0)),
        pl.BlockSpec((HID_CH, HID_CH), lambda i: (0, 0)),
        pl.BlockSpec((1, HID_CH), lambda i: (0, 0)),
    ],
    out_specs=pl.BlockSpec((BM, HID_CH), lambda i: (i, 0)),
    out_shape=jax.ShapeDtypeStruct((N, HID_CH), jnp.float32),
)

_d1_call = pl.pallas_call(
    _d1_body,
    grid=(_GM,),
    in_specs=[
        pl.BlockSpec((2, BM, IN_CH), lambda i: (0, i, 0)),
        pl.BlockSpec((BM, 1), lambda i: (i, 0)),
        pl.BlockSpec((BM, HID_CH), lambda i: (i, 0)),
        pl.BlockSpec((IN_CH, HID_CH), lambda i: (0, 0)),
    ],
    out_specs=pl.BlockSpec((2, BM, IN_CH), lambda i: (0, i, 0)),
    out_shape=jax.ShapeDtypeStruct((2, N, IN_CH), jnp.float32),
)

_d2_call = pl.pallas_call(
    _d2_body,
    grid=(_GM,),
    in_specs=[
        pl.BlockSpec((2, BM, IN_CH), lambda i: (0, i, 0)),
        pl.BlockSpec((BM, 1), lambda i: (i, 0)),
        pl.BlockSpec((BM, HID_CH), lambda i: (i, 0)),
        pl.BlockSpec((HID_CH, HID_CH), lambda i: (0, 0)),
        pl.BlockSpec((HID_CH, OUT_CH), lambda i: (0, 0)),
        pl.BlockSpec((1, OUT_CH), lambda i: (0, 0)),
    ],
    out_specs=pl.BlockSpec((BM, OUT_CH), lambda i: (i, 0)),
    out_shape=jax.ShapeDtypeStruct((N, OUT_CH), jnp.float32),
)


def kernel(x, edge_index, Wl1, bl1, Wr1, Wl2, bl2, Wr2, Wfc, bfc):
    src = edge_index[0].astype(jnp.int32)
    dst = edge_index[1].astype(jnp.int32)
    sd = jnp.stack([src.reshape(-1, K1), dst.reshape(-1, K1)], axis=1)
    sd3 = sd.reshape(NW, C1 // 4, 4, 2, K1)
    sdB = sd.reshape(NS, C2 // 4, 4, 2, K2)
    zf = jnp.zeros((ROWS_T, IN_CH), jnp.float32)
    z1 = jnp.zeros((ROWS_T,), jnp.float32)

    sc_layer1, sc_layer2 = _sc_kernels()
    xr = _root_call(x, Wr1, bl1.reshape(1, -1))
    aggp, cntp = sc_layer1(x, sd3, zf, z1)
    rcnt = (1.0 / jnp.clip(cntp[0] + cntp[1], 1.0, None))[:, None]
    h1s = _d1_call(aggp, rcnt, xr, Wl1)
    hr = _root2_call(h1s, Wr2, bl2.reshape(1, -1))
    agg2 = sc_layer2(h1s, sdB, zf)
    out = _d2_call(agg2, rcnt, hr, Wl2, Wfc, bfc.reshape(1, -1))
    return out


# speedup vs baseline: 1.0832x; 1.0109x over previous
import functools

import jax
import jax.numpy as jnp
from jax import lax
from jax.experimental import pallas as pl
from jax.experimental.pallas import tpu as pltpu
from jax.experimental.pallas import tpu_sc as plsc

N = 10000
E = 320000
IN_CH = 128
HID_CH = 256
OUT_CH = 64

NC = 2
NS = 16
NW = NC * NS

K1 = 125
C1 = E // NW // K1
K2 = 125
C2 = E // NS // K2
NP = 10240
ROWS_T = NP // NS

def _edge_pipeline(C, ld4, wl4, ig, wg, isc, wsc):
    M = C // 4
    assert C % 8 == 0 and M >= 4
    ld4(0, 0)
    wl4(0)
    ig(0, 0, 0)
    ig(1, 0, 1)
    wg(0, 0, 0)
    isc(0, 0, 0)
    wsc(0, 0, 0)
    ld4(1, 1)
    ig(0, 0, 2)
    wg(1, 0, 1)
    isc(1, 0, 1)
    wsc(1, 0, 1)
    ig(1, 0, 3)
    wg(0, 0, 2)
    isc(0, 0, 2)

    def superstep(m, p, prefetch):
        o = 1 - p
        wl4(p)
        wsc(0, o, 2)
        ig(0, p, 0)
        wg(1, o, 3)
        isc(1, o, 3)
        wsc(1, o, 3)
        ig(1, p, 1)
        wg(0, p, 0)
        isc(0, p, 0)
        wsc(0, p, 0)
        if prefetch:
            ld4(m + 1, o)
        ig(0, p, 2)
        wg(1, p, 1)
        isc(1, p, 1)
        wsc(1, p, 1)
        ig(1, p, 3)
        wg(0, p, 2)
        isc(0, p, 2)

    def body(t, carry):
        superstep(2 * t + 1, 1, True)
        superstep(2 * t + 2, 0, True)
        return carry

    lax.fori_loop(0, (M - 2) // 2, body, 0)
    superstep(M - 1, 1, False)
    wg(1, 1, 3)
    isc(1, 1, 3)
    wsc(0, 1, 2)
    wsc(1, 1, 3)


@functools.lru_cache(maxsize=None)
def _sc_kernels():
    mesh = plsc.VectorSubcoreMesh(core_axis_name="c", subcore_axis_name="s",
                                  num_cores=NC, num_subcores=NS)

    @functools.partial(
        pl.kernel,
        out_type=(
            jax.ShapeDtypeStruct((NC, NP, IN_CH), jnp.float32),
            jax.ShapeDtypeStruct((NC, NP), jnp.float32),
        ),
        mesh=mesh,
        scratch_types=[
            pltpu.VMEM((4, K1), jnp.int32),
            pltpu.VMEM((4, K1), jnp.int32),
            pltpu.VMEM((4, K1), jnp.int32),
            pltpu.VMEM((4, K1), jnp.int32),
            pltpu.VMEM((K1, IN_CH), jnp.float32),
            pltpu.VMEM((K1, IN_CH), jnp.float32),
            pltpu.VMEM((128,), jnp.float32),
            pltpu.VMEM_SHARED((NP, IN_CH), jnp.float32),
            pltpu.VMEM_SHARED((NP,), jnp.float32),
            pltpu.SemaphoreType.DMA,
            pltpu.SemaphoreType.DMA,
            pltpu.SemaphoreType.DMA,
            pltpu.SemaphoreType.DMA,
            pltpu.SemaphoreType.DMA,
            pltpu.SemaphoreType.DMA,
        ],
    )
    def sc_layer1(x_hbm, src4, dst4, zf, z1, aggp, cntp,
                  sb0, sb1, db0, db1, rv0, rv1, onesv, acc, cacc,
                  gs0, gs1, ss0, ss1, ip0, ip1):
        c = lax.axis_index("c")
        s = lax.axis_index("s")
        w = c * NS + s
        r0 = s * ROWS_T
        sb = (sb0, sb1)
        db = (db0, db1)
        rv = (rv0, rv1)
        gs = (gs0, gs1)
        ss = (ss0, ss1)
        pltpu.sync_copy(zf, acc.at[pl.ds(r0, ROWS_T)])
        pltpu.sync_copy(z1, cacc.at[pl.ds(r0, ROWS_T)])
        ones16 = jnp.ones((16,), jnp.float32)
        for i in range(8):
            onesv[pl.ds(i * 16, 16)] = ones16
        plsc.subcore_barrier()

        ip = (ip0, ip1)

        def ld4(m, p):
            pltpu.async_copy(src4.at[w].at[m], sb[p], ip[p])
            pltpu.async_copy(dst4.at[w].at[m], db[p], ip[p])

        def wl4(p):
            pltpu.make_async_copy(src4.at[w].at[0], sb[p], ip[p]).wait()
            pltpu.make_async_copy(dst4.at[w].at[0], db[p], ip[p]).wait()

        def ig(b, p, r):
            pltpu.async_copy(x_hbm.at[sb[p].at[r]], rv[b], gs[b])

        def wg(b, p, r):
            pltpu.make_async_copy(x_hbm.at[sb[p].at[r]], rv[b],
                                  gs[b]).wait()

        def isc(b, p, r):
            pltpu.async_copy(rv[b], acc.at[db[p].at[r]], ss[b],
                             add=True)
            pltpu.async_copy(onesv.at[pl.ds(0, K1)],
                             cacc.at[db[p].at[r]], ss[b], add=True)

        def wsc(b, p, r):
            pltpu.make_async_copy(rv[b], acc.at[db[p].at[r]],
                                  ss[b]).wait()
            pltpu.make_async_copy(onesv.at[pl.ds(0, K1)],
                                  cacc.at[db[p].at[r]], ss[b]).wait()

        _edge_pipeline(C1, ld4, wl4, ig, wg, isc, wsc)
        plsc.subcore_barrier()
        pltpu.sync_copy(acc.at[pl.ds(r0, ROWS_T)],
                        aggp.at[c].at[pl.ds(r0, ROWS_T)])
        pltpu.sync_copy(cacc.at[pl.ds(r0, ROWS_T)],
                        cntp.at[c].at[pl.ds(r0, ROWS_T)])

    @functools.partial(
        pl.kernel,
        out_type=jax.ShapeDtypeStruct((NC, NP, IN_CH), jnp.float32),
        mesh=mesh,
        scratch_types=[
            pltpu.VMEM((4, K2), jnp.int32),
            pltpu.VMEM((4, K2), jnp.int32),
            pltpu.VMEM((4, K2), jnp.int32),
            pltpu.VMEM((4, K2), jnp.int32),
            pltpu.VMEM((K2, IN_CH), jnp.float32),
            pltpu.VMEM((K2, IN_CH), jnp.float32),
            pltpu.VMEM_SHARED((NP, IN_CH), jnp.float32),
            pltpu.SemaphoreType.DMA,
            pltpu.SemaphoreType.DMA,
            pltpu.SemaphoreType.DMA,
            pltpu.SemaphoreType.DMA,
            pltpu.SemaphoreType.DMA,
            pltpu.SemaphoreType.DMA,
        ],
    )
    def sc_layer2(h1s_hbm, srcB, dstB, zf, agg2,
                  sb0, sb1, db0, db1, rv0, rv1, acc,
                  gs0, gs1, ss0, ss1, ip0, ip1):
        c = lax.axis_index("c")
        s = lax.axis_index("s")
        r0 = s * ROWS_T
        sb = (sb0, sb1)
        db = (db0, db1)
        rv = (rv0, rv1)
        gs = (gs0, gs1)
        ss = (ss0, ss1)
        pltpu.sync_copy(zf, acc.at[pl.ds(r0, ROWS_T)])
        plsc.subcore_barrier()

        ip = (ip0, ip1)

        def ld4(m, p):
            pltpu.async_copy(srcB.at[s].at[m], sb[p], ip[p])
            pltpu.async_copy(dstB.at[s].at[m], db[p], ip[p])

        def wl4(p):
            pltpu.make_async_copy(srcB.at[s].at[0], sb[p], ip[p]).wait()
            pltpu.make_async_copy(dstB.at[s].at[0], db[p], ip[p]).wait()

        def ig(b, p, r):
            pltpu.async_copy(h1s_hbm.at[c].at[sb[p].at[r]], rv[b],
                             gs[b])

        def wg(b, p, r):
            pltpu.make_async_copy(h1s_hbm.at[c].at[sb[p].at[r]],
                                  rv[b], gs[b]).wait()

        def isc(b, p, r):
            pltpu.async_copy(rv[b], acc.at[db[p].at[r]], ss[b],
                             add=True)

        def wsc(b, p, r):
            pltpu.make_async_copy(rv[b], acc.at[db[p].at[r]],
                                  ss[b]).wait()

        _edge_pipeline(C2, ld4, wl4, ig, wg, isc, wsc)
        plsc.subcore_barrier()
        pltpu.sync_copy(acc.at[pl.ds(r0, ROWS_T)],
                        agg2.at[c].at[pl.ds(r0, ROWS_T)])

    return sc_layer1, sc_layer2


BM = 2000
_GM = N // BM
_PREC = lax.Precision.HIGHEST


def _root_body(x, wr, bl, xr):
    xr[...] = lax.dot(x[...], wr[...], precision=_PREC) + bl[...]


def _root2_body(h1s, wr, bl, hr):
    h1 = jnp.concatenate([h1s[0], h1s[1]], axis=1)
    hr[...] = lax.dot(h1, wr[...], precision=_PREC) + bl[...]


def _d1_body(aggp, rcnt, xr, wl, h1s):
    agg = aggp[0] + aggp[1]
    mw = lax.dot(agg, wl[...], precision=_PREC) * rcnt[...]
    h = jnp.maximum(mw + xr[...], 0.0)
    h1s[0] = h[:, :IN_CH]
    h1s[1] = h[:, IN_CH:]


def _d2_body(agg2, rcnt, hr, wl, wfc, bfc, out):
    agg = jnp.concatenate([agg2[0], agg2[1]], axis=1)
    mw = lax.dot(agg, wl[...], precision=_PREC) * rcnt[...]
    h2 = jnp.maximum(mw + hr[...], 0.0)
    z = lax.dot(h2, wfc[...], precision=_PREC) + bfc[...]
    m = jnp.max(z, axis=1, keepdims=True)
    e = z - m
    out[...] = e - jnp.log(jnp.sum(jnp.exp(e), axis=1, keepdims=True))


_root_call = pl.pallas_call(
    _root_body,
    grid=(_GM,),
    in_specs=[
        pl.BlockSpec((BM, IN_CH), lambda i: (i, 0)),
        pl.BlockSpec((IN_CH, HID_CH), lambda i: (0, 0)),
        pl.BlockSpec((1, HID_CH), lambda i: (0, 0)),
    ],
    out_specs=pl.BlockSpec((BM, HID_CH), lambda i: (i, 0)),
    out_shape=jax.ShapeDtypeStruct((N, HID_CH), jnp.float32),
)

_root2_call = pl.pallas_call(
    _root2_body,
    grid=(_GM,),
    in_specs=[
        pl.BlockSpec((2, BM, IN_CH), lambda i: (0, i, 0)),
        pl.BlockSpec((HID_CH, HID_CH), lambda i: (0, 0)),
        pl.BlockSpec((1, HID_CH), lambda i: (0, 0)),
    ],
    out_specs=pl.BlockSpec((BM, HID_CH), lambda i: (i, 0)),
    out_shape=jax.ShapeDtypeStruct((N, HID_CH), jnp.float32),
)

_d1_call = pl.pallas_call(
    _d1_body,
    grid=(_GM,),
    in_specs=[
        pl.BlockSpec((2, BM, IN_CH), lambda i: (0, i, 0)),
        pl.BlockSpec((BM, 1), lambda i: (i, 0)),
        pl.BlockSpec((BM, HID_CH), lambda i: (i, 0)),
        pl.BlockSpec((IN_CH, HID_CH), lambda i: (0, 0)),
    ],
    out_specs=pl.BlockSpec((2, BM, IN_CH), lambda i: (0, i, 0)),
    out_shape=jax.ShapeDtypeStruct((2, N, IN_CH), jnp.float32),
)

_d2_call = pl.pallas_call(
    _d2_body,
    grid=(_GM,),
    in_specs=[
        pl.BlockSpec((2, BM, IN_CH), lambda i: (0, i, 0)),
        pl.BlockSpec((BM, 1), lambda i: (i, 0)),
        pl.BlockSpec((BM, HID_CH), lambda i: (i, 0)),
        pl.BlockSpec((HID_CH, HID_CH), lambda i: (0, 0)),
        pl.BlockSpec((HID_CH, OUT_CH), lambda i: (0, 0)),
        pl.BlockSpec((1, OUT_CH), lambda i: (0, 0)),
    ],
    out_specs=pl.BlockSpec((BM, OUT_CH), lambda i: (i, 0)),
    out_shape=jax.ShapeDtypeStruct((N, OUT_CH), jnp.float32),
)


def kernel(x, edge_index, Wl1, bl1, Wr1, Wl2, bl2, Wr2, Wfc, bfc):
    src = edge_index[0].astype(jnp.int32)
    dst = edge_index[1].astype(jnp.int32)
    src4 = src.reshape(NW, C1 // 4, 4, K1)
    dst4 = dst.reshape(NW, C1 // 4, 4, K1)
    srcB = src.reshape(NS, C2 // 4, 4, K2)
    dstB = dst.reshape(NS, C2 // 4, 4, K2)
    zf = jnp.zeros((ROWS_T, IN_CH), jnp.float32)
    z1 = jnp.zeros((ROWS_T,), jnp.float32)

    sc_layer1, sc_layer2 = _sc_kernels()
    xr = _root_call(x, Wr1, bl1.reshape(1, -1))
    aggp, cntp = sc_layer1(x, src4, dst4, zf, z1)
    rcnt = (1.0 / jnp.clip(cntp[0] + cntp[1], 1.0, None))[:, None]
    h1s = _d1_call(aggp, rcnt, xr, Wl1)
    hr = _root2_call(h1s, Wr2, bl2.reshape(1, -1))
    agg2 = sc_layer2(h1s, srcB, dstB, zf)
    out = _d2_call(agg2, rcnt, hr, Wl2, Wfc, bfc.reshape(1, -1))
    return out


# speedup vs baseline: 1.0851x; 1.0017x over previous
import functools

import jax
import jax.numpy as jnp
from jax import lax
from jax.experimental import pallas as pl
from jax.experimental.pallas import tpu as pltpu
from jax.experimental.pallas import tpu_sc as plsc

N = 10000
E = 320000
IN_CH = 128
HID_CH = 256
OUT_CH = 64

NC = 2
NS = 16
NW = NC * NS

K1 = 125
C1 = E // NW // K1
K2 = 125
C2 = E // NS // K2
NP = 10240
ROWS_T = NP // NS

def _edge_pipeline(C, ld4, wl4, ig, wg, isc, wsc):
    M = C // 4
    assert C % 8 == 0 and M >= 4
    ld4(0, 0)
    wl4(0)
    ig(0, 0, 0)
    ig(1, 0, 1)
    wg(0, 0, 0)
    isc(0, 0, 0)
    wsc(0, 0, 0)
    ld4(1, 1)
    ig(0, 0, 2)
    wg(1, 0, 1)
    isc(1, 0, 1)
    wsc(1, 0, 1)
    ig(1, 0, 3)
    wg(0, 0, 2)
    isc(0, 0, 2)

    def superstep(m, p, prefetch):
        o = 1 - p
        wl4(p)
        wsc(0, o, 2)
        ig(0, p, 0)
        wg(1, o, 3)
        isc(1, o, 3)
        wsc(1, o, 3)
        ig(1, p, 1)
        wg(0, p, 0)
        isc(0, p, 0)
        wsc(0, p, 0)
        if prefetch:
            ld4(m + 1, o)
        ig(0, p, 2)
        wg(1, p, 1)
        isc(1, p, 1)
        wsc(1, p, 1)
        ig(1, p, 3)
        wg(0, p, 2)
        isc(0, p, 2)

    def body(t, carry):
        superstep(2 * t + 1, 1, True)
        superstep(2 * t + 2, 0, True)
        return carry

    lax.fori_loop(0, (M - 2) // 2, body, 0)
    superstep(M - 1, 1, False)
    wg(1, 1, 3)
    isc(1, 1, 3)
    wsc(0, 1, 2)
    wsc(1, 1, 3)


@functools.lru_cache(maxsize=None)
def _sc_kernels():
    mesh = plsc.VectorSubcoreMesh(core_axis_name="c", subcore_axis_name="s",
                                  num_cores=NC, num_subcores=NS)

    @functools.partial(
        pl.kernel,
        out_type=(
            jax.ShapeDtypeStruct((NC, NP, IN_CH), jnp.float32),
            jax.ShapeDtypeStruct((NC, NP), jnp.float32),
        ),
        mesh=mesh,
        scratch_types=[
            pltpu.VMEM((4, K1), jnp.int32),
            pltpu.VMEM((4, K1), jnp.int32),
            pltpu.VMEM((4, K1), jnp.int32),
            pltpu.VMEM((4, K1), jnp.int32),
            pltpu.VMEM((K1, IN_CH), jnp.float32),
            pltpu.VMEM((K1, IN_CH), jnp.float32),
            pltpu.VMEM((128,), jnp.float32),
            pltpu.VMEM_SHARED((NP, IN_CH), jnp.float32),
            pltpu.VMEM_SHARED((NP,), jnp.float32),
            pltpu.SemaphoreType.DMA,
            pltpu.SemaphoreType.DMA,
            pltpu.SemaphoreType.DMA,
            pltpu.SemaphoreType.DMA,
            pltpu.SemaphoreType.DMA,
            pltpu.SemaphoreType.DMA,
        ],
    )
    def sc_layer1(x_hbm, src4, dst4, zf, z1, aggp, cntp,
                  sb0, sb1, db0, db1, rv0, rv1, onesv, acc, cacc,
                  gs0, gs1, ss0, ss1, ip0, ip1):
        c = lax.axis_index("c")
        s = lax.axis_index("s")
        w = c * NS + s
        r0 = s * ROWS_T
        sb = (sb0, sb1)
        db = (db0, db1)
        rv = (rv0, rv1)
        gs = (gs0, gs1)
        ss = (ss0, ss1)
        pltpu.sync_copy(zf, acc.at[pl.ds(r0, ROWS_T)])
        pltpu.sync_copy(z1, cacc.at[pl.ds(r0, ROWS_T)])
        ones16 = jnp.ones((16,), jnp.float32)
        for i in range(8):
            onesv[pl.ds(i * 16, 16)] = ones16
        plsc.subcore_barrier()

        ip = (ip0, ip1)

        def ld4(m, p):
            pltpu.async_copy(src4.at[w].at[m], sb[p], ip[p])
            pltpu.async_copy(dst4.at[w].at[m], db[p], ip[p])

        def wl4(p):
            pltpu.make_async_copy(src4.at[w].at[0], sb[p], ip[p]).wait()
            pltpu.make_async_copy(dst4.at[w].at[0], db[p], ip[p]).wait()

        def ig(b, p, r):
            pltpu.async_copy(x_hbm.at[sb[p].at[r]], rv[b], gs[b])

        def wg(b, p, r):
            pltpu.make_async_copy(x_hbm.at[sb[p].at[r]], rv[b],
                                  gs[b]).wait()

        def isc(b, p, r):
            pltpu.async_copy(rv[b], acc.at[db[p].at[r]], ss[b],
                             add=True)
            pltpu.async_copy(onesv.at[pl.ds(0, K1)],
                             cacc.at[db[p].at[r]], ss[b], add=True)

        def wsc(b, p, r):
            pltpu.make_async_copy(rv[b], acc.at[db[p].at[r]],
                                  ss[b]).wait()
            pltpu.make_async_copy(onesv.at[pl.ds(0, K1)],
                                  cacc.at[db[p].at[r]], ss[b]).wait()

        _edge_pipeline(C1, ld4, wl4, ig, wg, isc, wsc)
        plsc.subcore_barrier()
        pltpu.sync_copy(acc.at[pl.ds(r0, ROWS_T)],
                        aggp.at[c].at[pl.ds(r0, ROWS_T)])
        pltpu.sync_copy(cacc.at[pl.ds(r0, ROWS_T)],
                        cntp.at[c].at[pl.ds(r0, ROWS_T)])

    @functools.partial(
        pl.kernel,
        out_type=jax.ShapeDtypeStruct((NC, NP, IN_CH), jnp.float32),
        mesh=mesh,
        scratch_types=[
            pltpu.VMEM((4, K2), jnp.int32),
            pltpu.VMEM((4, K2), jnp.int32),
            pltpu.VMEM((4, K2), jnp.int32),
            pltpu.VMEM((4, K2), jnp.int32),
            pltpu.VMEM((K2, IN_CH), jnp.float32),
            pltpu.VMEM((K2, IN_CH), jnp.float32),
            pltpu.VMEM_SHARED((NP, IN_CH), jnp.float32),
            pltpu.SemaphoreType.DMA,
            pltpu.SemaphoreType.DMA,
            pltpu.SemaphoreType.DMA,
            pltpu.SemaphoreType.DMA,
            pltpu.SemaphoreType.DMA,
            pltpu.SemaphoreType.DMA,
        ],
    )
    def sc_layer2(h1s_hbm, src4, dst4, zf, agg2,
                  sb0, sb1, db0, db1, rv0, rv1, acc,
                  gs0, gs1, ss0, ss1, ip0, ip1):
        c = lax.axis_index("c")
        s = lax.axis_index("s")
        r0 = s * ROWS_T
        sb = (sb0, sb1)
        db = (db0, db1)
        rv = (rv0, rv1)
        gs = (gs0, gs1)
        ss = (ss0, ss1)
        pltpu.sync_copy(zf, acc.at[pl.ds(r0, ROWS_T)])
        plsc.subcore_barrier()

        ip = (ip0, ip1)

        def ld4(m, p):
            w2 = 2 * s + m // (C1 // 4)
            mm = m % (C1 // 4)
            pltpu.async_copy(src4.at[w2].at[mm], sb[p], ip[p])
            pltpu.async_copy(dst4.at[w2].at[mm], db[p], ip[p])

        def wl4(p):
            pltpu.make_async_copy(src4.at[0].at[0], sb[p], ip[p]).wait()
            pltpu.make_async_copy(dst4.at[0].at[0], db[p], ip[p]).wait()

        def ig(b, p, r):
            pltpu.async_copy(h1s_hbm.at[c].at[sb[p].at[r]], rv[b],
                             gs[b])

        def wg(b, p, r):
            pltpu.make_async_copy(h1s_hbm.at[c].at[sb[p].at[r]],
                                  rv[b], gs[b]).wait()

        def isc(b, p, r):
            pltpu.async_copy(rv[b], acc.at[db[p].at[r]], ss[b],
                             add=True)

        def wsc(b, p, r):
            pltpu.make_async_copy(rv[b], acc.at[db[p].at[r]],
                                  ss[b]).wait()

        _edge_pipeline(C2, ld4, wl4, ig, wg, isc, wsc)
        plsc.subcore_barrier()
        pltpu.sync_copy(acc.at[pl.ds(r0, ROWS_T)],
                        agg2.at[c].at[pl.ds(r0, ROWS_T)])

    return sc_layer1, sc_layer2


BM = 2000
_GM = N // BM
_PREC = lax.Precision.HIGHEST


def _root_body(x, wr, bl, xr):
    xr[...] = lax.dot(x[...], wr[...], precision=_PREC) + bl[...]


def _root2_body(h1s, wr, bl, hr):
    h1 = jnp.concatenate([h1s[0], h1s[1]], axis=1)
    hr[...] = lax.dot(h1, wr[...], precision=_PREC) + bl[...]


def _d1_body(aggp, rcnt, xr, wl, h1s):
    agg = aggp[0] + aggp[1]
    mw = lax.dot(agg, wl[...], precision=_PREC) * rcnt[...]
    h = jnp.maximum(mw + xr[...], 0.0)
    h1s[0] = h[:, :IN_CH]
    h1s[1] = h[:, IN_CH:]


def _d2_body(agg2, rcnt, hr, wl, wfc, bfc, out):
    agg = jnp.concatenate([agg2[0], agg2[1]], axis=1)
    mw = lax.dot(agg, wl[...], precision=_PREC) * rcnt[...]
    h2 = jnp.maximum(mw + hr[...], 0.0)
    z = lax.dot(h2, wfc[...], precision=_PREC) + bfc[...]
    m = jnp.max(z, axis=1, keepdims=True)
    e = z - m
    out[...] = e - jnp.log(jnp.sum(jnp.exp(e), axis=1, keepdims=True))


_root_call = pl.pallas_call(
    _root_body,
    grid=(_GM,),
    in_specs=[
        pl.BlockSpec((BM, IN_CH), lambda i: (i, 0)),
        pl.BlockSpec((IN_CH, HID_CH), lambda i: (0, 0)),
        pl.BlockSpec((1, HID_CH), lambda i: (0, 0)),
    ],
    out_specs=pl.BlockSpec((BM, HID_CH), lambda i: (i, 0)),
    out_shape=jax.ShapeDtypeStruct((N, HID_CH), jnp.float32),
)

_root2_call = pl.pallas_call(
    _root2_body,
    grid=(_GM,),
    in_specs=[
        pl.BlockSpec((2, BM, IN_CH), lambda i: (0, i, 0)),
        pl.BlockSpec((HID_CH, HID_CH), lambda i: (0, 0)),
        pl.BlockSpec((1, HID_CH), lambda i: (0, 0)),
    ],
    out_specs=pl.BlockSpec((BM, HID_CH), lambda i: (i, 0)),
    out_shape=jax.ShapeDtypeStruct((N, HID_CH), jnp.float32),
)

_d1_call = pl.pallas_call(
    _d1_body,
    grid=(_GM,),
    in_specs=[
        pl.BlockSpec((2, BM, IN_CH), lambda i: (0, i, 0)),
        pl.BlockSpec((BM, 1), lambda i: (i, 0)),
        pl.BlockSpec((BM, HID_CH), lambda i: (i, 0)),
        pl.BlockSpec((IN_CH, HID_CH), lambda i: (0, 0)),
    ],
    out_specs=pl.BlockSpec((2, BM, IN_CH), lambda i: (0, i, 0)),
    out_shape=jax.ShapeDtypeStruct((2, N, IN_CH), jnp.float32),
)

_d2_call = pl.pallas_call(
    _d2_body,
    grid=(_GM,),
    in_specs=[
        pl.BlockSpec((2, BM, IN_CH), lambda i: (0, i, 0)),
        pl.BlockSpec((BM, 1), lambda i: (i, 0)),
        pl.BlockSpec((BM, HID_CH), lambda i: (i, 0)),
        pl.BlockSpec((HID_CH, HID_CH), lambda i: (0, 0)),
        pl.BlockSpec((HID_CH, OUT_CH), lambda i: (0, 0)),
        pl.BlockSpec((1, OUT_CH), lambda i: (0, 0)),
    ],
    out_specs=pl.BlockSpec((BM, OUT_CH), lambda i: (i, 0)),
    out_shape=jax.ShapeDtypeStruct((N, OUT_CH), jnp.float32),
)


def kernel(x, edge_index, Wl1, bl1, Wr1, Wl2, bl2, Wr2, Wfc, bfc):
    src = edge_index[0].astype(jnp.int32)
    dst = edge_index[1].astype(jnp.int32)
    src4 = src.reshape(NW, C1 // 4, 4, K1)
    dst4 = dst.reshape(NW, C1 // 4, 4, K1)
    zf = jnp.zeros((ROWS_T, IN_CH), jnp.float32)
    z1 = jnp.zeros((ROWS_T,), jnp.float32)

    sc_layer1, sc_layer2 = _sc_kernels()
    xr = _root_call(x, Wr1, bl1.reshape(1, -1))
    aggp, cntp = sc_layer1(x, src4, dst4, zf, z1)
    rcnt = (1.0 / jnp.clip(cntp[0] + cntp[1], 1.0, None))[:, None]
    h1s = _d1_call(aggp, rcnt, xr, Wl1)
    hr = _root2_call(h1s, Wr2, bl2.reshape(1, -1))
    agg2 = sc_layer2(h1s, src4, dst4, zf)
    out = _d2_call(agg2, rcnt, hr, Wl2, Wfc, bfc.reshape(1, -1))
    return out


# speedup vs baseline: 1.1155x; 1.0280x over previous
import functools

import jax
import jax.numpy as jnp
from jax import lax
from jax.experimental import pallas as pl
from jax.experimental.pallas import tpu as pltpu
from jax.experimental.pallas import tpu_sc as plsc

N = 10000
E = 320000
IN_CH = 128
HID_CH = 256
OUT_CH = 64

NC = 2
NS = 16
NW = NC * NS

K1 = 125
C1 = E // NW // K1
K2 = 125
C2 = E // NS // K2
NP = 10240
ROWS_T = NP // NS

def _edge_pipeline(C, ld4, wl4, ig, wg, isc, wsc):
    M = C // 4
    assert C % 8 == 0 and M >= 4
    ld4(0, 0)
    wl4(0)
    ig(0, 0, 0)
    ig(1, 0, 1)
    wg(0, 0, 0)
    isc(0, 0, 0)
    wsc(0, 0, 0)
    ld4(1, 1)
    ig(0, 0, 2)
    wg(1, 0, 1)
    isc(1, 0, 1)
    wsc(1, 0, 1)
    ig(1, 0, 3)
    wg(0, 0, 2)
    isc(0, 0, 2)

    def superstep(m, p, prefetch):
        o = 1 - p
        wl4(p)
        wsc(0, o, 2)
        ig(0, p, 0)
        wg(1, o, 3)
        isc(1, o, 3)
        wsc(1, o, 3)
        ig(1, p, 1)
        wg(0, p, 0)
        isc(0, p, 0)
        wsc(0, p, 0)
        if prefetch:
            ld4(m + 1, o)
        ig(0, p, 2)
        wg(1, p, 1)
        isc(1, p, 1)
        wsc(1, p, 1)
        ig(1, p, 3)
        wg(0, p, 2)
        isc(0, p, 2)

    def body(t, carry):
        superstep(2 * t + 1, 1, True)
        superstep(2 * t + 2, 0, True)
        return carry

    lax.fori_loop(0, (M - 2) // 2, body, 0)
    superstep(M - 1, 1, False)
    wg(1, 1, 3)
    isc(1, 1, 3)
    wsc(0, 1, 2)
    wsc(1, 1, 3)


@functools.lru_cache(maxsize=None)
def _sc_kernels():
    mesh = plsc.VectorSubcoreMesh(core_axis_name="c", subcore_axis_name="s",
                                  num_cores=NC, num_subcores=NS)

    @functools.partial(
        pl.kernel,
        out_type=(
            jax.ShapeDtypeStruct((NC, NP, IN_CH), jnp.float32),
            jax.ShapeDtypeStruct((NC, NP), jnp.float32),
        ),
        mesh=mesh,
        scratch_types=[
            pltpu.VMEM((4, K1), jnp.int32),
            pltpu.VMEM((4, K1), jnp.int32),
            pltpu.VMEM((4, K1), jnp.int32),
            pltpu.VMEM((4, K1), jnp.int32),
            pltpu.VMEM((K1, IN_CH), jnp.float32),
            pltpu.VMEM((K1, IN_CH), jnp.float32),
            pltpu.VMEM((128,), jnp.float32),
            pltpu.VMEM_SHARED((NP, IN_CH), jnp.float32),
            pltpu.VMEM_SHARED((NP,), jnp.float32),
            pltpu.SemaphoreType.DMA,
            pltpu.SemaphoreType.DMA,
            pltpu.SemaphoreType.DMA,
            pltpu.SemaphoreType.DMA,
            pltpu.SemaphoreType.DMA,
            pltpu.SemaphoreType.DMA,
        ],
    )
    def sc_layer1(x_hbm, ei5, zf, z1, aggp, cntp,
                  sb0, sb1, db0, db1, rv0, rv1, onesv, acc, cacc,
                  gs0, gs1, ss0, ss1, ip0, ip1):
        c = lax.axis_index("c")
        s = lax.axis_index("s")
        w = c * NS + s
        r0 = s * ROWS_T
        sb = (sb0, sb1)
        db = (db0, db1)
        rv = (rv0, rv1)
        gs = (gs0, gs1)
        ss = (ss0, ss1)
        pltpu.sync_copy(zf, acc.at[pl.ds(r0, ROWS_T)])
        pltpu.sync_copy(z1, cacc.at[pl.ds(r0, ROWS_T)])
        ones16 = jnp.ones((16,), jnp.float32)
        for i in range(8):
            onesv[pl.ds(i * 16, 16)] = ones16
        plsc.subcore_barrier()

        ip = (ip0, ip1)

        def ld4(m, p):
            pltpu.async_copy(ei5.at[0].at[w].at[m], sb[p], ip[p])
            pltpu.async_copy(ei5.at[1].at[w].at[m], db[p], ip[p])

        def wl4(p):
            pltpu.make_async_copy(ei5.at[0].at[0].at[0], sb[p], ip[p]).wait()
            pltpu.make_async_copy(ei5.at[1].at[0].at[0], db[p], ip[p]).wait()

        def ig(b, p, r):
            pltpu.async_copy(x_hbm.at[sb[p].at[r]], rv[b], gs[b])

        def wg(b, p, r):
            pltpu.make_async_copy(x_hbm.at[sb[p].at[r]], rv[b],
                                  gs[b]).wait()

        def isc(b, p, r):
            pltpu.async_copy(rv[b], acc.at[db[p].at[r]], ss[b],
                             add=True)
            pltpu.async_copy(onesv.at[pl.ds(0, K1)],
                             cacc.at[db[p].at[r]], ss[b], add=True)

        def wsc(b, p, r):
            pltpu.make_async_copy(rv[b], acc.at[db[p].at[r]],
                                  ss[b]).wait()
            pltpu.make_async_copy(onesv.at[pl.ds(0, K1)],
                                  cacc.at[db[p].at[r]], ss[b]).wait()

        _edge_pipeline(C1, ld4, wl4, ig, wg, isc, wsc)
        plsc.subcore_barrier()
        pltpu.sync_copy(acc.at[pl.ds(r0, ROWS_T)],
                        aggp.at[c].at[pl.ds(r0, ROWS_T)])
        pltpu.sync_copy(cacc.at[pl.ds(r0, ROWS_T)],
                        cntp.at[c].at[pl.ds(r0, ROWS_T)])

    @functools.partial(
        pl.kernel,
        out_type=jax.ShapeDtypeStruct((NC, NP, IN_CH), jnp.float32),
        mesh=mesh,
        scratch_types=[
            pltpu.VMEM((4, K2), jnp.int32),
            pltpu.VMEM((4, K2), jnp.int32),
            pltpu.VMEM((4, K2), jnp.int32),
            pltpu.VMEM((4, K2), jnp.int32),
            pltpu.VMEM((K2, IN_CH), jnp.float32),
            pltpu.VMEM((K2, IN_CH), jnp.float32),
            pltpu.VMEM_SHARED((NP, IN_CH), jnp.float32),
            pltpu.SemaphoreType.DMA,
            pltpu.SemaphoreType.DMA,
            pltpu.SemaphoreType.DMA,
            pltpu.SemaphoreType.DMA,
            pltpu.SemaphoreType.DMA,
            pltpu.SemaphoreType.DMA,
        ],
    )
    def sc_layer2(h1s_hbm, ei5, zf, agg2,
                  sb0, sb1, db0, db1, rv0, rv1, acc,
                  gs0, gs1, ss0, ss1, ip0, ip1):
        c = lax.axis_index("c")
        s = lax.axis_index("s")
        r0 = s * ROWS_T
        sb = (sb0, sb1)
        db = (db0, db1)
        rv = (rv0, rv1)
        gs = (gs0, gs1)
        ss = (ss0, ss1)
        pltpu.sync_copy(zf, acc.at[pl.ds(r0, ROWS_T)])
        plsc.subcore_barrier()

        ip = (ip0, ip1)

        def ld4(m, p):
            w2 = 2 * s + m // (C1 // 4)
            mm = m % (C1 // 4)
            pltpu.async_copy(ei5.at[0].at[w2].at[mm], sb[p], ip[p])
            pltpu.async_copy(ei5.at[1].at[w2].at[mm], db[p], ip[p])

        def wl4(p):
            pltpu.make_async_copy(ei5.at[0].at[0].at[0], sb[p], ip[p]).wait()
            pltpu.make_async_copy(ei5.at[1].at[0].at[0], db[p], ip[p]).wait()

        def ig(b, p, r):
            pltpu.async_copy(h1s_hbm.at[c].at[sb[p].at[r]], rv[b],
                             gs[b])

        def wg(b, p, r):
            pltpu.make_async_copy(h1s_hbm.at[c].at[sb[p].at[r]],
                                  rv[b], gs[b]).wait()

        def isc(b, p, r):
            pltpu.async_copy(rv[b], acc.at[db[p].at[r]], ss[b],
                             add=True)

        def wsc(b, p, r):
            pltpu.make_async_copy(rv[b], acc.at[db[p].at[r]],
                                  ss[b]).wait()

        _edge_pipeline(C2, ld4, wl4, ig, wg, isc, wsc)
        plsc.subcore_barrier()
        pltpu.sync_copy(acc.at[pl.ds(r0, ROWS_T)],
                        agg2.at[c].at[pl.ds(r0, ROWS_T)])

    return sc_layer1, sc_layer2


BM = 2000
_GM = N // BM
_PREC = lax.Precision.HIGHEST


def _root_body(x, wr, bl, xr):
    xr[...] = lax.dot(x[...], wr[...], precision=_PREC) + bl[...]


def _root2_body(h1s, wr, bl, hr):
    h1 = jnp.concatenate([h1s[0], h1s[1]], axis=1)
    hr[...] = lax.dot(h1, wr[...], precision=_PREC) + bl[...]


def _d1_body(aggp, rcnt, xr, wl, h1s):
    agg = aggp[0] + aggp[1]
    mw = lax.dot(agg, wl[...], precision=_PREC) * rcnt[...]
    h = jnp.maximum(mw + xr[...], 0.0)
    h1s[0] = h[:, :IN_CH]
    h1s[1] = h[:, IN_CH:]


def _d2_body(agg2, rcnt, hr, wl, wfc, bfc, out):
    agg = jnp.concatenate([agg2[0], agg2[1]], axis=1)
    mw = lax.dot(agg, wl[...], precision=_PREC) * rcnt[...]
    h2 = jnp.maximum(mw + hr[...], 0.0)
    z = lax.dot(h2, wfc[...], precision=_PREC) + bfc[...]
    m = jnp.max(z, axis=1, keepdims=True)
    e = z - m
    out[...] = e - jnp.log(jnp.sum(jnp.exp(e), axis=1, keepdims=True))


_root_call = pl.pallas_call(
    _root_body,
    grid=(_GM,),
    in_specs=[
        pl.BlockSpec((BM, IN_CH), lambda i: (i, 0)),
        pl.BlockSpec((IN_CH, HID_CH), lambda i: (0, 0)),
        pl.BlockSpec((1, HID_CH), lambda i: (0, 0)),
    ],
    out_specs=pl.BlockSpec((BM, HID_CH), lambda i: (i, 0)),
    out_shape=jax.ShapeDtypeStruct((N, HID_CH), jnp.float32),
)

_root2_call = pl.pallas_call(
    _root2_body,
    grid=(_GM,),
    in_specs=[
        pl.BlockSpec((2, BM, IN_CH), lambda i: (0, i, 0)),
        pl.BlockSpec((HID_CH, HID_CH), lambda i: (0, 0)),
        pl.BlockSpec((1, HID_CH), lambda i: (0, 0)),
    ],
    out_specs=pl.BlockSpec((BM, HID_CH), lambda i: (i, 0)),
    out_shape=jax.ShapeDtypeStruct((N, HID_CH), jnp.float32),
)

_d1_call = pl.pallas_call(
    _d1_body,
    grid=(_GM,),
    in_specs=[
        pl.BlockSpec((2, BM, IN_CH), lambda i: (0, i, 0)),
        pl.BlockSpec((BM, 1), lambda i: (i, 0)),
        pl.BlockSpec((BM, HID_CH), lambda i: (i, 0)),
        pl.BlockSpec((IN_CH, HID_CH), lambda i: (0, 0)),
    ],
    out_specs=pl.BlockSpec((2, BM, IN_CH), lambda i: (0, i, 0)),
    out_shape=jax.ShapeDtypeStruct((2, N, IN_CH), jnp.float32),
)

_d2_call = pl.pallas_call(
    _d2_body,
    grid=(_GM,),
    in_specs=[
        pl.BlockSpec((2, BM, IN_CH), lambda i: (0, i, 0)),
        pl.BlockSpec((BM, 1), lambda i: (i, 0)),
        pl.BlockSpec((BM, HID_CH), lambda i: (i, 0)),
        pl.BlockSpec((HID_CH, HID_CH), lambda i: (0, 0)),
        pl.BlockSpec((HID_CH, OUT_CH), lambda i: (0, 0)),
        pl.BlockSpec((1, OUT_CH), lambda i: (0, 0)),
    ],
    out_specs=pl.BlockSpec((BM, OUT_CH), lambda i: (i, 0)),
    out_shape=jax.ShapeDtypeStruct((N, OUT_CH), jnp.float32),
)


def kernel(x, edge_index, Wl1, bl1, Wr1, Wl2, bl2, Wr2, Wfc, bfc):
    ei5 = edge_index.astype(jnp.int32).reshape(2, NW, C1 // 4, 4, K1)
    zf = jnp.zeros((ROWS_T, IN_CH), jnp.float32)
    z1 = jnp.zeros((ROWS_T,), jnp.float32)

    sc_layer1, sc_layer2 = _sc_kernels()
    xr = _root_call(x, Wr1, bl1.reshape(1, -1))
    aggp, cntp = sc_layer1(x, ei5, zf, z1)
    rcnt = (1.0 / jnp.clip(cntp[0] + cntp[1], 1.0, None))[:, None]
    h1s = _d1_call(aggp, rcnt, xr, Wl1)
    hr = _root2_call(h1s, Wr2, bl2.reshape(1, -1))
    agg2 = sc_layer2(h1s, ei5, zf)
    out = _d2_call(agg2, rcnt, hr, Wl2, Wfc, bfc.reshape(1, -1))
    return out
